# baseline (speedup 1.0000x reference)
import jax
import jax.numpy as jnp
from jax.experimental import pallas as pl
from jax.experimental.pallas import tpu as pltpu

B, T, D = 2, 4096, 2048
E = 64
TOP_K = 2
TILE = 256
NTILES = (B * T) // TILE
Q = 8

_NEG_INF = float("-inf")


def _top2_softmax(logits):
    idx = jax.lax.broadcasted_iota(jnp.int32, logits.shape, 1)
    m1 = jnp.max(logits, axis=-1, keepdims=True)
    i1 = jnp.min(jnp.where(logits == m1, idx, E), axis=-1, keepdims=True)
    masked = jnp.where(idx == i1, _NEG_INF, logits)
    m2 = jnp.max(masked, axis=-1, keepdims=True)
    i2 = jnp.min(jnp.where(masked == m2, idx, E), axis=-1, keepdims=True)
    e2 = jnp.exp(m2 - m1)
    denom = 1.0 + e2
    weights = jnp.concatenate([1.0 / denom, e2 / denom], axis=-1)
    indices = jnp.concatenate([i1, i2], axis=-1)
    return weights, indices


def _router_kernel(x_hbm, w_ref, logits_ref, weights_ref, indices_ref,
                   xbuf, sems):
    i = pl.program_id(0)

    def copy(step, slot):
        return pltpu.make_async_copy(
            x_hbm.at[pl.ds(step * TILE, TILE), :],
            xbuf.at[slot],
            sems.at[slot],
        )

    @pl.when(i == 0)
    def _():
        for q in range(Q):
            copy(q, q).start()

    slot = jax.lax.rem(i, Q)
    copy(i, slot).wait()

    logits = jax.lax.dot_general(
        xbuf[slot], w_ref[...], (((1,), (1,)), ((), ())),
        preferred_element_type=jnp.float32,
    )
    logits_ref[...] = logits
    weights, indices = _top2_softmax(logits)
    weights_ref[...] = weights
    indices_ref[...] = indices

    @pl.when(i + Q < NTILES)
    def _():
        copy(i + Q, slot).start()


@jax.jit
def kernel(x, W):
    xt = x.reshape(B * T, D)
    logits, weights, indices = pl.pallas_call(
        _router_kernel,
        grid=(NTILES,),
        in_specs=[
            pl.BlockSpec(memory_space=pltpu.MemorySpace.HBM),
            pl.BlockSpec((E, D), lambda i: (0, 0)),
        ],
        out_specs=[
            pl.BlockSpec((TILE, E), lambda i: (i, 0)),
            pl.BlockSpec((TILE, TOP_K), lambda i: (i, 0)),
            pl.BlockSpec((TILE, TOP_K), lambda i: (i, 0)),
        ],
        out_shape=[
            jax.ShapeDtypeStruct((B * T, E), jnp.float32),
            jax.ShapeDtypeStruct((B * T, TOP_K), jnp.float32),
            jax.ShapeDtypeStruct((B * T, TOP_K), jnp.int32),
        ],
        scratch_shapes=[
            pltpu.VMEM((Q, TILE, D), jnp.float32),
            pltpu.SemaphoreType.DMA((Q,)),
        ],
        compiler_params=pltpu.CompilerParams(
            dimension_semantics=("arbitrary",),
        ),
    )(xt, W)
    return (
        weights.reshape(B, T, TOP_K),
        indices.reshape(B, T, TOP_K),
        logits.reshape(B, T, E),
    )


# speedup vs baseline: 1.1685x; 1.1685x over previous
import jax
import jax.numpy as jnp
from jax.experimental import pallas as pl
from jax.experimental.pallas import tpu as pltpu

B, T, D = 2, 4096, 2048
E = 64
TOP_K = 2
TILE = 1024
NTILES = (B * T) // TILE
Q = 4

_NEG_INF = float("-inf")


def _top2_softmax(logits):
    idx = jax.lax.broadcasted_iota(jnp.int32, logits.shape, 1)
    m1 = jnp.max(logits, axis=-1, keepdims=True)
    i1 = jnp.min(jnp.where(logits == m1, idx, E), axis=-1, keepdims=True)
    masked = jnp.where(idx == i1, _NEG_INF, logits)
    m2 = jnp.max(masked, axis=-1, keepdims=True)
    i2 = jnp.min(jnp.where(masked == m2, idx, E), axis=-1, keepdims=True)
    e2 = jnp.exp(m2 - m1)
    denom = 1.0 + e2
    weights = jnp.concatenate([1.0 / denom, e2 / denom], axis=-1)
    indices = jnp.concatenate([i1, i2], axis=-1)
    return weights, indices


def _router_kernel(x_hbm, w_ref, logits_ref, weights_ref, indices_ref,
                   xbuf, sems):
    i = pl.program_id(0)

    def copy(step, slot):
        return pltpu.make_async_copy(
            x_hbm.at[pl.ds(step * TILE, TILE), :],
            xbuf.at[slot],
            sems.at[slot],
        )

    @pl.when(i == 0)
    def _():
        for q in range(Q):
            copy(q, q).start()

    slot = jax.lax.rem(i, Q)
    copy(i, slot).wait()

    logits = jax.lax.dot_general(
        xbuf[slot], w_ref[...], (((1,), (1,)), ((), ())),
        preferred_element_type=jnp.float32,
    )
    logits_ref[...] = logits
    weights, indices = _top2_softmax(logits)
    weights_ref[...] = weights
    indices_ref[...] = indices

    @pl.when(i + Q < NTILES)
    def _():
        copy(i + Q, slot).start()


@jax.jit
def kernel(x, W):
    xt = x.reshape(B * T, D)
    logits, weights, indices = pl.pallas_call(
        _router_kernel,
        grid=(NTILES,),
        in_specs=[
            pl.BlockSpec(memory_space=pltpu.MemorySpace.HBM),
            pl.BlockSpec((E, D), lambda i: (0, 0)),
        ],
        out_specs=[
            pl.BlockSpec((TILE, E), lambda i: (i, 0)),
            pl.BlockSpec((TILE, TOP_K), lambda i: (i, 0)),
            pl.BlockSpec((TILE, TOP_K), lambda i: (i, 0)),
        ],
        out_shape=[
            jax.ShapeDtypeStruct((B * T, E), jnp.float32),
            jax.ShapeDtypeStruct((B * T, TOP_K), jnp.float32),
            jax.ShapeDtypeStruct((B * T, TOP_K), jnp.int32),
        ],
        scratch_shapes=[
            pltpu.VMEM((Q, TILE, D), jnp.float32),
            pltpu.SemaphoreType.DMA((Q,)),
        ],
        compiler_params=pltpu.CompilerParams(
            dimension_semantics=("arbitrary",),
        ),
    )(xt, W)
    return (
        weights.reshape(B, T, TOP_K),
        indices.reshape(B, T, TOP_K),
        logits.reshape(B, T, E),
    )


# speedup vs baseline: 1.2079x; 1.0337x over previous
import jax
import jax.numpy as jnp
from jax.experimental import pallas as pl
from jax.experimental.pallas import tpu as pltpu

B, T, D = 2, 4096, 2048
E = 64
TOP_K = 2
TILE = 512
NTILES = (B * T) // TILE
Q = 4

_NEG_INF = float("-inf")


def _top2_softmax(logits):
    idx = jax.lax.broadcasted_iota(jnp.int32, logits.shape, 1)
    m1 = jnp.max(logits, axis=-1, keepdims=True)
    i1 = jnp.min(jnp.where(logits == m1, idx, E), axis=-1, keepdims=True)
    masked = jnp.where(idx == i1, _NEG_INF, logits)
    m2 = jnp.max(masked, axis=-1, keepdims=True)
    i2 = jnp.min(jnp.where(masked == m2, idx, E), axis=-1, keepdims=True)
    e2 = jnp.exp(m2 - m1)
    denom = 1.0 + e2
    weights = jnp.concatenate([1.0 / denom, e2 / denom], axis=-1)
    indices = jnp.concatenate([i1, i2], axis=-1)
    return weights, indices


def _router_kernel(x_hbm, w_ref, logits_ref, weights_ref, indices_ref,
                   xbuf, sems):
    i = pl.program_id(0)

    def copy(step, slot, h):
        half = TILE // 2
        return pltpu.make_async_copy(
            x_hbm.at[pl.ds(step * TILE + h * half, half), :],
            xbuf.at[slot, pl.ds(h * half, half), :],
            sems.at[slot, h],
        )

    @pl.when(i == 0)
    def _():
        for q in range(Q):
            copy(q, q, 0).start()
            copy(q, q, 1).start()

    slot = jax.lax.rem(i, Q)
    copy(i, slot, 0).wait()
    copy(i, slot, 1).wait()

    logits = jax.lax.dot_general(
        xbuf[slot], w_ref[...], (((1,), (1,)), ((), ())),
        preferred_element_type=jnp.float32,
    )
    logits_ref[...] = logits
    weights, indices = _top2_softmax(logits)
    weights_ref[...] = weights
    indices_ref[...] = indices

    @pl.when(i + Q < NTILES)
    def _():
        copy(i + Q, slot, 0).start()
        copy(i + Q, slot, 1).start()


@jax.jit
def kernel(x, W):
    xt = x.reshape(B * T, D)
    logits, weights, indices = pl.pallas_call(
        _router_kernel,
        grid=(NTILES,),
        in_specs=[
            pl.BlockSpec(memory_space=pltpu.MemorySpace.HBM),
            pl.BlockSpec((E, D), lambda i: (0, 0)),
        ],
        out_specs=[
            pl.BlockSpec((TILE, E), lambda i: (i, 0)),
            pl.BlockSpec((TILE, TOP_K), lambda i: (i, 0)),
            pl.BlockSpec((TILE, TOP_K), lambda i: (i, 0)),
        ],
        out_shape=[
            jax.ShapeDtypeStruct((B * T, E), jnp.float32),
            jax.ShapeDtypeStruct((B * T, TOP_K), jnp.float32),
            jax.ShapeDtypeStruct((B * T, TOP_K), jnp.int32),
        ],
        scratch_shapes=[
            pltpu.VMEM((Q, TILE, D), jnp.float32),
            pltpu.SemaphoreType.DMA((Q, 2)),
        ],
        compiler_params=pltpu.CompilerParams(
            dimension_semantics=("arbitrary",),
        ),
    )(xt, W)
    return (
        weights.reshape(B, T, TOP_K),
        indices.reshape(B, T, TOP_K),
        logits.reshape(B, T, E),
    )
